# baseline (speedup 1.0000x reference)
import functools

import jax
import jax.numpy as jnp
from jax import lax
from jax.experimental import pallas as pl
from jax.experimental.pallas import tpu as pltpu
from jax.experimental.pallas import tpu_sc as plsc

B = 16384
D = 32
NC, NS = 2, 16
NW = NC * NS
BPW = B // NW
NG = BPW // 16


def _sc_gather_body(idx_hbm, user_hbm, item_hbm, uout_hbm, iout_hbm,
                    idx_v, chunk_v, stage_v, sems, osems):
    wid = lax.axis_index("s") * NC + lax.axis_index("c")
    base = wid * BPW
    pltpu.sync_copy(idx_hbm.at[wid], idx_v)
    rvec = lax.iota(jnp.int32, 16)

    for tb, (tab, out) in enumerate(((user_hbm, uout_hbm),
                                     (item_hbm, iout_hbm))):
        row0 = tb * 4

        def load_vec(g, row0=row0):
            return idx_v[row0 + g // 8, pl.ds((g % 8) * 16, 16)]

        def fire(i, slot, tab=tab):
            t = lax.shift_right_logical(i, 7)
            pltpu.async_copy(tab.at[:, pl.ds(t * 128, 128)],
                             chunk_v.at[slot], sems.at[slot])

        def drain(slot, tab=tab):
            pltpu.make_async_copy(tab.at[:, pl.ds(0, 128)],
                                  chunk_v.at[slot], sems.at[slot]).wait()

        def extract(i, slot, lane, half):
            cvec = rvec * 0 + jnp.bitwise_and(i, 127)
            lo = plsc.load_gather(chunk_v.at[slot, pl.ds(0, 16)], [rvec, cvec])
            hi = plsc.load_gather(chunk_v.at[slot, pl.ds(16, 16)], [rvec, cvec])
            stage_v[half, lane, pl.ds(0, 16)] = lo
            stage_v[half, lane, pl.ds(16, 16)] = hi

        def flush(g, half, out=out):
            pltpu.async_copy(stage_v.at[half],
                             out.at[pl.ds(base + g * 16, 16)], osems.at[half])

        def wait_flush(half, tab=tab):
            pltpu.make_async_copy(tab.at[pl.ds(0, 16), pl.ds(0, 128)],
                                  stage_v.at[half], osems.at[half]).wait()

        def make_sub(half):
            def sub(g, pvec, do_wait):
                if do_wait:
                    wait_flush(half)
                gvec = load_vec(g)
                for lane in range(16):
                    drain(lane)
                    extract(pvec[lane], lane, lane, half)
                    fire(gvec[lane], lane)
                flush(g - 1, half)
                return gvec
            return sub

        sub0 = make_sub(0)
        sub1 = make_sub(1)

        vec0 = load_vec(0)
        for lane in range(16):
            fire(vec0[lane], lane)
        v = sub0(1, vec0, False)
        v = sub1(2, v, False)

        def body(p, pvec):
            va = sub0(2 * p - 1, pvec, True)
            return sub1(2 * p, va, True)

        v = lax.fori_loop(2, NG // 2, body, v)
        v = sub0(NG - 1, v, True)
        wait_flush(1)
        for lane in range(16):
            drain(lane)
            extract(v[lane], lane, lane, 1)
        flush(NG - 1, 1)
        wait_flush(0)
        wait_flush(1)


@functools.cache
def _sc_gather():
    mesh = plsc.VectorSubcoreMesh(
        core_axis_name="c", subcore_axis_name="s", num_cores=NC, num_subcores=NS
    )
    return pl.kernel(
        _sc_gather_body,
        out_type=(
            jax.ShapeDtypeStruct((B, 128), jnp.float32),
            jax.ShapeDtypeStruct((B, 128), jnp.float32),
        ),
        mesh=mesh,
        scratch_types=[
            pltpu.VMEM((8, 128), jnp.int32),
            pltpu.VMEM((16, 32, 128), jnp.float32),
            pltpu.VMEM((2, 16, 128), jnp.float32),
            pltpu.SemaphoreType.DMA((16,)),
            pltpu.SemaphoreType.DMA((2,)),
        ],
        compiler_params=pltpu.CompilerParams(
            use_tc_tiling_on_sc=True, needs_layout_passes=False),
    )


def _mlp_body(u_ref, i_ref, wu_ref, wi_ref, bc_ref, wb_ref, out_ref):
    h = jnp.dot(u_ref[:, :D], wu_ref[...], preferred_element_type=jnp.float32)
    h = h + jnp.dot(i_ref[:, :D], wi_ref[...],
                    preferred_element_type=jnp.float32)
    h = jnp.maximum(h + bc_ref[...], 0.0)
    out_ref[...] = jnp.dot(h, wb_ref[...], preferred_element_type=jnp.float32)


_BS = 2048


def _mlp(u, i, wu, wi, bc, wb):
    return pl.pallas_call(
        _mlp_body,
        grid=(B // _BS,),
        in_specs=[
            pl.BlockSpec((_BS, 128), lambda j: (j, 0)),
            pl.BlockSpec((_BS, 128), lambda j: (j, 0)),
            pl.BlockSpec((D, 2 * D), lambda j: (0, 0)),
            pl.BlockSpec((D, 2 * D), lambda j: (0, 0)),
            pl.BlockSpec((1, 2 * D), lambda j: (0, 0)),
            pl.BlockSpec((2 * D, 2), lambda j: (0, 0)),
        ],
        out_specs=pl.BlockSpec((_BS, 2), lambda j: (j, 0)),
        out_shape=jax.ShapeDtypeStruct((B, 2), jnp.float32),
    )(u, i, wu, wi, bc, wb)


def kernel(x, user_emb, item_emb, W1a, b1a, W1b, W0a, b0a, W0b):
    xi = x.astype(jnp.int32)
    idx = xi.reshape(NW, 4, 128, 2).transpose(0, 3, 1, 2).reshape(NW, 8, 128)
    uraw, iraw = _sc_gather()(idx, user_emb.T, item_emb.T)

    wu = jnp.concatenate([W1a[:, :D], W0a[:, :D]], axis=0).T
    wi = jnp.concatenate([W1a[:, D:], W0a[:, D:]], axis=0).T
    bc = jnp.concatenate([b1a, b0a]).reshape(1, 2 * D)
    wb = jnp.zeros((2 * D, 2), jnp.float32)
    wb = wb.at[:D, 0].set(W1b[0]).at[D:, 1].set(W0b[0])

    y = _mlp(uraw, iraw, wu, wi, bc, wb)
    return (y[:, 0:1], y[:, 1:2])


# speedup vs baseline: 1.0378x; 1.0378x over previous
import functools

import jax
import jax.numpy as jnp
from jax import lax
from jax.experimental import pallas as pl
from jax.experimental.pallas import tpu as pltpu
from jax.experimental.pallas import tpu_sc as plsc

B = 16384
D = 32
NC, NS = 2, 16
NW = NC * NS
BPW = B // NW
NG = BPW // 16


def _sc_gather_body(idx_hbm, user_hbm, item_hbm, uout_hbm, iout_hbm,
                    idx_v, chunk_v, stage_v, sems, osem):
    wid = lax.axis_index("s") * NC + lax.axis_index("c")
    base = wid * BPW
    pltpu.sync_copy(idx_hbm.at[wid], idx_v)
    rvec = lax.iota(jnp.int32, 16)

    for tb, (tab, out) in enumerate(((user_hbm, uout_hbm),
                                     (item_hbm, iout_hbm))):
        row0 = tb * 4

        def load_vec(g, row0=row0):
            return idx_v[row0 + g // 8, pl.ds((g % 8) * 16, 16)]

        def fire(i, slot, tab=tab):
            t = lax.shift_right_logical(i, 7)
            pltpu.async_copy(tab.at[:, pl.ds(t * 128, 128)],
                             chunk_v.at[slot], sems.at[slot])

        def drain(slot, tab=tab):
            pltpu.make_async_copy(tab.at[:, pl.ds(0, 128)],
                                  chunk_v.at[slot], sems.at[slot]).wait()

        def extract(i, slot, lane):
            cvec = rvec * 0 + jnp.bitwise_and(i, 127)
            lo = plsc.load_gather(chunk_v.at[slot, pl.ds(0, 16)], [rvec, cvec])
            hi = plsc.load_gather(chunk_v.at[slot, pl.ds(16, 16)], [rvec, cvec])
            stage_v[lane, pl.ds(0, 16)] = lo
            stage_v[lane, pl.ds(16, 16)] = hi

        def flush(g, out=out):
            pltpu.async_copy(stage_v, out.at[pl.ds(base + g * 16, 16)],
                             osem).wait()

        vec0 = load_vec(0)
        for lane in range(16):
            fire(vec0[lane], lane)

        def body(g, pvec):
            gvec = load_vec(g)
            for lane in range(16):
                drain(lane)
                extract(pvec[lane], lane, lane)
                fire(gvec[lane], lane)
            flush(g - 1)
            return gvec

        pvec = lax.fori_loop(1, NG, body, vec0)
        for lane in range(16):
            drain(lane)
            extract(pvec[lane], lane, lane)
        flush(NG - 1)


@functools.cache
def _sc_gather():
    mesh = plsc.VectorSubcoreMesh(
        core_axis_name="c", subcore_axis_name="s", num_cores=NC, num_subcores=NS
    )
    return pl.kernel(
        _sc_gather_body,
        out_type=(
            jax.ShapeDtypeStruct((B, 128), jnp.float32),
            jax.ShapeDtypeStruct((B, 128), jnp.float32),
        ),
        mesh=mesh,
        scratch_types=[
            pltpu.VMEM((8, 128), jnp.int32),
            pltpu.VMEM((16, 32, 128), jnp.float32),
            pltpu.VMEM((16, 128), jnp.float32),
            pltpu.SemaphoreType.DMA((16,)),
            pltpu.SemaphoreType.DMA,
        ],
        compiler_params=pltpu.CompilerParams(
            use_tc_tiling_on_sc=True, needs_layout_passes=False),
    )


def _mlp_body(u_ref, i_ref, wu_ref, wi_ref, bc_ref, wb_ref, y1_ref, y0_ref):
    h = jnp.dot(u_ref[:, :D], wu_ref[...], preferred_element_type=jnp.float32)
    h = h + jnp.dot(i_ref[:, :D], wi_ref[...],
                    preferred_element_type=jnp.float32)
    h = jnp.maximum(h + bc_ref[...], 0.0)
    y = jnp.dot(h, wb_ref[...], preferred_element_type=jnp.float32)
    y1_ref[...] = y[:, 0:1]
    y0_ref[...] = y[:, 1:2]


_BS = 2048


def _mlp(u, i, wu, wi, bc, wb):
    return pl.pallas_call(
        _mlp_body,
        grid=(B // _BS,),
        in_specs=[
            pl.BlockSpec((_BS, 128), lambda j: (j, 0)),
            pl.BlockSpec((_BS, 128), lambda j: (j, 0)),
            pl.BlockSpec((D, 2 * D), lambda j: (0, 0)),
            pl.BlockSpec((D, 2 * D), lambda j: (0, 0)),
            pl.BlockSpec((1, 2 * D), lambda j: (0, 0)),
            pl.BlockSpec((2 * D, 2), lambda j: (0, 0)),
        ],
        out_specs=[
            pl.BlockSpec((_BS, 1), lambda j: (j, 0)),
            pl.BlockSpec((_BS, 1), lambda j: (j, 0)),
        ],
        out_shape=(
            jax.ShapeDtypeStruct((B, 1), jnp.float32),
            jax.ShapeDtypeStruct((B, 1), jnp.float32),
        ),
    )(u, i, wu, wi, bc, wb)


def kernel(x, user_emb, item_emb, W1a, b1a, W1b, W0a, b0a, W0b):
    xi = x.astype(jnp.int32)
    idx = xi.reshape(NW, 4, 128, 2).transpose(0, 3, 1, 2).reshape(NW, 8, 128)
    uraw, iraw = _sc_gather()(idx, user_emb.T, item_emb.T)

    wu = jnp.concatenate([W1a[:, :D], W0a[:, :D]], axis=0).T
    wi = jnp.concatenate([W1a[:, D:], W0a[:, D:]], axis=0).T
    bc = jnp.concatenate([b1a, b0a]).reshape(1, 2 * D)
    wb = jnp.zeros((2 * D, 2), jnp.float32)
    wb = wb.at[:D, 0].set(W1b[0]).at[D:, 1].set(W0b[0])

    y1, y0 = _mlp(uraw, iraw, wu, wi, bc, wb)
    return (y1, y0)
